# Initial kernel scaffold; baseline (speedup 1.0000x reference)
import functools

import jax
import jax.numpy as jnp
from jax import lax
from jax.experimental import pallas as pl
from jax.experimental.pallas import tpu as pltpu
from jax.experimental.pallas import tpu_sc as plsc

NUM_EMB = 1_000_000
D = 64
B = 16384
L = 200

NC = 2
NS = 16
NW = NC * NS

ROWS_PER_W = B // NW
G = 4
NG = ROWS_PER_W // G
IDX_PER_G = G * L
N_SLICE = (IDX_PER_G + 127) // 128


def _body(kw_hbm, table_hbm, out_hbm,
          idx0, idx1, rows0, rows1, outb0, outb1,
          sem_g0, sem_g1, sem_i0, sem_i1, sem_o0, sem_o1):
    wid = lax.axis_index("s") * NC + lax.axis_index("c")
    idx_base = wid * (ROWS_PER_W * L)
    out_base = wid * ROWS_PER_W

    bufs = ((idx0, rows0, outb0, sem_g0, sem_i0, sem_o0),
            (idx1, rows1, outb1, sem_g1, sem_i1, sem_o1))

    def fire_idx(g, p):
        idx_v, _, _, _, sem_i, _ = bufs[p]
        pltpu.async_copy(kw_hbm.at[pl.ds(idx_base + g * IDX_PER_G, IDX_PER_G)],
                         idx_v, sem_i)

    def wait_idx(p):
        idx_v, _, _, _, sem_i, _ = bufs[p]
        pltpu.make_async_copy(kw_hbm.at[pl.ds(idx_base, IDX_PER_G)],
                              idx_v, sem_i).wait()

    def fire_gathers(p):
        idx_v, rows_v, _, sem_g, _, _ = bufs[p]
        for j in range(N_SLICE):
            ln = min(128, IDX_PER_G - j * 128)
            pltpu.async_copy(
                table_hbm.at[idx_v.at[pl.ds(j * 128, ln)]],
                rows_v.at[pl.ds(j * 128, ln)],
                sem_g)

    def wait_gathers(p):
        _, rows_v, _, sem_g, _, _ = bufs[p]
        pltpu.make_async_copy(table_hbm.at[pl.ds(0, IDX_PER_G)],
                              rows_v, sem_g).wait()

    def fire_out(g, p):
        _, _, out_v, _, _, sem_o = bufs[p]
        pltpu.async_copy(out_v, out_hbm.at[pl.ds(out_base + g * G, G)], sem_o)

    def wait_out(p):
        _, _, out_v, _, _, sem_o = bufs[p]
        pltpu.make_async_copy(out_v, out_hbm.at[pl.ds(out_base, G)],
                              sem_o).wait()

    def accumulate(p):
        _, rows_v, out_v, _, _, _ = bufs[p]
        z = jnp.zeros((16,), jnp.float32)
        for r in range(G):
            base = r * L

            def acc_body(j, accs, base=base):
                return tuple(accs[d] + rows_v[base + j, pl.ds(d * 16, 16)]
                             for d in range(4))

            a = lax.fori_loop(0, L, acc_body, (z, z, z, z), unroll=8)
            for d in range(4):
                out_v[r, pl.ds(d * 16, 16)] = a[d]

    fire_idx(0, 0)
    fire_idx(1, 1)
    wait_idx(0)
    fire_gathers(0)
    wait_idx(1)
    fire_gathers(1)

    def group_iter(i, carry):
        for p in range(2):
            g = 2 * i + p
            wait_gathers(p)

            @pl.when(g < NG - 2)
            def _fire_next_idx(g=g, p=p):
                fire_idx(g + 2, p)

            @pl.when(g >= 2)
            def _wait_prev_out(p=p):
                wait_out(p)

            accumulate(p)
            fire_out(g, p)

            @pl.when(g < NG - 2)
            def _fire_next_gathers(p=p):
                wait_idx(p)
                fire_gathers(p)
        return carry

    lax.fori_loop(0, NG // 2, group_iter, 0)

    wait_out(0)
    wait_out(1)


@jax.jit
def kernel(kw, table):
    kw_flat = kw.reshape(-1).astype(jnp.int32)
    mesh = plsc.VectorSubcoreMesh(core_axis_name="c", subcore_axis_name="s")
    k = pl.kernel(
        _body,
        out_type=jax.ShapeDtypeStruct((B, D), jnp.float32),
        mesh=mesh,
        scratch_types=[
            pltpu.VMEM((IDX_PER_G,), jnp.int32),
            pltpu.VMEM((IDX_PER_G,), jnp.int32),
            pltpu.VMEM((IDX_PER_G, D), jnp.float32),
            pltpu.VMEM((IDX_PER_G, D), jnp.float32),
            pltpu.VMEM((G, D), jnp.float32),
            pltpu.VMEM((G, D), jnp.float32),
            pltpu.SemaphoreType.DMA,
            pltpu.SemaphoreType.DMA,
            pltpu.SemaphoreType.DMA,
            pltpu.SemaphoreType.DMA,
            pltpu.SemaphoreType.DMA,
            pltpu.SemaphoreType.DMA,
        ],
    )
    return k(kw_flat, table)


# speedup vs baseline: 3.3977x; 3.3977x over previous
import functools

import jax
import jax.numpy as jnp
from jax import lax
from jax.experimental import pallas as pl
from jax.experimental.pallas import tpu as pltpu
from jax.experimental.pallas import tpu_sc as plsc

NUM_EMB = 1_000_000
D = 64
B = 16384
L = 200

NC = 2
NS = 16
NW = NC * NS

ROWS_PER_W = B // NW
G = 4
NG = ROWS_PER_W // G
IDX_PER_G = G * L
N_SLICE = (IDX_PER_G + 127) // 128


def _body(kw_hbm, table_hbm, out_hbm,
          idx0, idx1, rows0, rows1, outb0, outb1,
          sem_g0, sem_g1, sem_i0, sem_i1, sem_o0, sem_o1):
    wid = lax.axis_index("s") * NC + lax.axis_index("c")
    idx_base = wid * (ROWS_PER_W * L)
    out_base = wid * ROWS_PER_W

    bufs = ((idx0, rows0, outb0, sem_g0, sem_i0, sem_o0),
            (idx1, rows1, outb1, sem_g1, sem_i1, sem_o1))

    def fire_idx(g, p):
        idx_v, _, _, _, sem_i, _ = bufs[p]
        pltpu.async_copy(kw_hbm.at[pl.ds(idx_base + g * IDX_PER_G, IDX_PER_G)],
                         idx_v, sem_i)

    def wait_idx(p):
        idx_v, _, _, _, sem_i, _ = bufs[p]
        pltpu.make_async_copy(kw_hbm.at[pl.ds(idx_base, IDX_PER_G)],
                              idx_v, sem_i).wait()

    def fire_gathers(p):
        idx_v, rows_v, _, sem_g, _, _ = bufs[p]
        for j in range(N_SLICE):
            ln = min(128, IDX_PER_G - j * 128)
            pltpu.async_copy(
                table_hbm.at[idx_v.at[pl.ds(j * 128, ln)]],
                rows_v.at[pl.ds(j * 128, ln)],
                sem_g)

    def wait_gathers(p):
        _, rows_v, _, sem_g, _, _ = bufs[p]
        pltpu.make_async_copy(table_hbm.at[pl.ds(0, IDX_PER_G)],
                              rows_v, sem_g).wait()

    def fire_out(g, p):
        _, _, out_v, _, _, sem_o = bufs[p]
        pltpu.async_copy(out_v, out_hbm.at[pl.ds(out_base + g * G, G)], sem_o)

    def wait_out(p):
        _, _, out_v, _, _, sem_o = bufs[p]
        pltpu.make_async_copy(out_v, out_hbm.at[pl.ds(out_base, G)],
                              sem_o).wait()

    def accumulate(p):
        _, rows_v, out_v, _, _, _ = bufs[p]
        z = jnp.zeros((16,), jnp.float32)
        for r in range(G):
            base = r * L

            def acc_body(j, accs, base=base):
                return tuple(accs[d] + rows_v[base + j, pl.ds(d * 16, 16)]
                             for d in range(4))

            a = lax.fori_loop(0, L, acc_body, (z, z, z, z), unroll=8)
            for d in range(4):
                out_v[r, pl.ds(d * 16, 16)] = a[d]

    fire_idx(0, 0)
    fire_idx(1, 1)
    wait_idx(0)
    fire_gathers(0)
    wait_idx(1)
    fire_gathers(1)

    def group_iter(i, carry):
        for p in range(2):
            g = 2 * i + p
            wait_gathers(p)

            @pl.when(g < NG - 2)
            def _fire_next_idx(g=g, p=p):
                fire_idx(g + 2, p)

            @pl.when(g >= 2)
            def _wait_prev_out(p=p):
                wait_out(p)

            accumulate(p)
            fire_out(g, p)

            @pl.when(g < NG - 2)
            def _fire_next_gathers(p=p):
                wait_idx(p)
                fire_gathers(p)
        return carry

    lax.fori_loop(0, NG // 2, group_iter, 0)

    wait_out(0)
    wait_out(1)


@jax.jit
def kernel(kw, table):
    kw_flat = kw.reshape(-1).astype(jnp.int32)
    mesh = plsc.VectorSubcoreMesh(core_axis_name="c", subcore_axis_name="s")
    k = pl.kernel(
        _body,
        out_type=jax.ShapeDtypeStruct((B, D), jnp.float32),
        mesh=mesh,
        compiler_params=pltpu.CompilerParams(use_tc_tiling_on_sc=False),
        scratch_types=[
            pltpu.VMEM((IDX_PER_G,), jnp.int32),
            pltpu.VMEM((IDX_PER_G,), jnp.int32),
            pltpu.VMEM((IDX_PER_G, D), jnp.float32),
            pltpu.VMEM((IDX_PER_G, D), jnp.float32),
            pltpu.VMEM((G, D), jnp.float32),
            pltpu.VMEM((G, D), jnp.float32),
            pltpu.SemaphoreType.DMA,
            pltpu.SemaphoreType.DMA,
            pltpu.SemaphoreType.DMA,
            pltpu.SemaphoreType.DMA,
            pltpu.SemaphoreType.DMA,
            pltpu.SemaphoreType.DMA,
        ],
    )
    return k(kw_flat, table)


# speedup vs baseline: 5.9029x; 1.7373x over previous
import jax
import jax.numpy as jnp
from jax import lax
from jax.experimental import pallas as pl
from jax.experimental.pallas import tpu as pltpu
from jax.experimental.pallas import tpu_sc as plsc

NUM_EMB = 1_000_000
D = 64
B = 16384
L = 200

NC = 2
NS = 16
NW = NC * NS

ROWS_PER_W = B // NW
G = 4
NG = ROWS_PER_W // G
IDX_PER_G = G * L
SLICE_OFFS = tuple((r * L + o, ln) for r in range(G) for o, ln in
                   ((0, 128), (128, L - 128)))
N_SLICE = len(SLICE_OFFS)


_SLICES_AFTER_ROW = tuple((2 * r, 2 * r + 1) for r in range(G))


def _body(kw_hbm, table_hbm, out_hbm,
          idx0, idx1, idx2, idx3, rows0, rows1, outb0, outb1,
          sem_i0, sem_i1, sem_i2, sem_i3,
          sem_g0, sem_g1, sem_o0, sem_o1):
    wid = lax.axis_index("s") * NC + lax.axis_index("c")
    idx_base = wid * (ROWS_PER_W * L)
    out_base = wid * ROWS_PER_W

    idxs = ((idx0, sem_i0), (idx1, sem_i1), (idx2, sem_i2), (idx3, sem_i3))
    rows = ((rows0, sem_g0), (rows1, sem_g1))
    outs = ((outb0, sem_o0), (outb1, sem_o1))

    def fire_idx(g, s):
        idx_v, sem_i = idxs[s]
        pltpu.async_copy(kw_hbm.at[pl.ds(idx_base + g * IDX_PER_G, IDX_PER_G)],
                         idx_v, sem_i)

    def wait_idx(s):
        idx_v, sem_i = idxs[s]
        pltpu.make_async_copy(kw_hbm.at[pl.ds(idx_base, IDX_PER_G)],
                              idx_v, sem_i).wait()

    def remap_idx(s):
        idx_v, _ = idxs[s]
        for i in range(IDX_PER_G // 16):
            v = idx_v[pl.ds(i * 16, 16)]
            m = ((v & (-TW)) + ((v & (TH - 1)) << 1)
                 + ((v >> (TH.bit_length() - 1)) & 1))
            idx_v[pl.ds(i * 16, 16)] = m

    def fire_gather_slice(s, p, j):
        idx_v, _ = idxs[s]
        rows_v, sem_g = rows[p]
        off, ln = SLICE_OFFS[j]
        pltpu.async_copy(
            table_hbm.at[idx_v.at[pl.ds(off, ln)]],
            rows_v.at[pl.ds(off, ln)],
            sem_g)

    def wait_gathers(p):
        rows_v, sem_g = rows[p]
        pltpu.make_async_copy(table_hbm.at[pl.ds(0, IDX_PER_G)],
                              rows_v, sem_g).wait()

    def fire_out(g, p):
        out_v, sem_o = outs[p]
        pltpu.async_copy(out_v, out_hbm.at[pl.ds(out_base + g * G, G)], sem_o)

    def wait_out(p):
        out_v, sem_o = outs[p]
        pltpu.make_async_copy(out_v, out_hbm.at[pl.ds(out_base, G)],
                              sem_o).wait()

    def accumulate_and_prefetch(g, p, s_next):
        rows_v, _ = rows[p]
        out_v, _ = outs[p]
        z = jnp.zeros((16,), jnp.float32)
        for r in range(G):
            base = r * L

            def acc_body(j, accs, base=base):
                return tuple(accs[d] + rows_v[base + j, pl.ds(d * 16, 16)]
                             for d in range(4))

            a = lax.fori_loop(0, L, acc_body, (z, z, z, z), unroll=8)
            for d in range(4):
                out_v[r, pl.ds(d * 16, 16)] = a[d]

            @pl.when(g < NG - 2)
            def _fire_slices(p=p, s_next=s_next, r=r):
                for j in _SLICES_AFTER_ROW[r]:
                    fire_gather_slice(s_next, p, j)

    fire_idx(0, 0)
    fire_idx(1, 1)
    fire_idx(2, 2)
    wait_idx(0)
    remap_idx(0)
    for j in range(N_SLICE):
        fire_gather_slice(0, 0, j)
    wait_idx(1)
    remap_idx(1)
    for j in range(N_SLICE):
        fire_gather_slice(1, 1, j)

    def group_iter(i, carry):
        for q in range(4):
            g = 4 * i + q
            p = q % 2
            wait_gathers(p)

            @pl.when(g < NG - 3)
            def _fire_next_idx(g=g, q=q):
                fire_idx(g + 3, (q + 3) % 4)

            @pl.when(g >= 2)
            def _wait_prev_out(p=p):
                wait_out(p)

            @pl.when(g < NG - 2)
            def _ready_next_idx(q=q):
                wait_idx((q + 2) % 4)
                remap_idx((q + 2) % 4)

            accumulate_and_prefetch(g, p, (q + 2) % 4)
            fire_out(g, p)
        return carry

    lax.fori_loop(0, NG // 4, group_iter, 0)

    wait_out(0)
    wait_out(1)


TW = 32768
TH = TW // 2


def _transpose_body(src_ref, dst_ref):
    dst_ref[:, 0:D] = src_ref[:, 0:TH].T
    dst_ref[:, D:128] = src_ref[:, TH:TW].T


def _relayout_table(table_t):
    n_blocks = (NUM_EMB + TW - 1) // TW
    return pl.pallas_call(
        _transpose_body,
        grid=(n_blocks,),
        in_specs=[pl.BlockSpec((D, TW), lambda j: (0, j))],
        out_specs=pl.BlockSpec((TH, 128), lambda j: (j, 0)),
        out_shape=jax.ShapeDtypeStruct((n_blocks * TH, 128), jnp.float32),
    )(table_t)


@jax.jit
def kernel(kw, table):
    kw_flat = kw.reshape(-1).astype(jnp.int32)
    tbl = _relayout_table(table.T)
    table_lin = tbl.reshape(tbl.shape[0] * 2, D)
    mesh = plsc.VectorSubcoreMesh(core_axis_name="c", subcore_axis_name="s")
    k = pl.kernel(
        _body,
        out_type=jax.ShapeDtypeStruct((B, D), jnp.float32),
        mesh=mesh,
        compiler_params=pltpu.CompilerParams(use_tc_tiling_on_sc=False),
        scratch_types=(
            [pltpu.VMEM((IDX_PER_G,), jnp.int32) for _ in range(4)]
            + [pltpu.VMEM((IDX_PER_G, D), jnp.float32) for _ in range(2)]
            + [pltpu.VMEM((G, D), jnp.float32) for _ in range(2)]
            + [pltpu.SemaphoreType.DMA for _ in range(8)]
        ),
    )
    return k(kw_flat, table_lin)
